# baseline (speedup 1.0000x reference)
import jax
import jax.numpy as jnp
from jax import lax
from jax.experimental import pallas as pl
from jax.experimental.pallas import tpu as pltpu
from jax.experimental.pallas import tpu_sc as plsc

N = 10000
DEG = 32
F = 128
NPG = 4
IPG = NPG * DEG
NGROUPS = N // NPG
NWORKERS = 32
GPW = 80
MAX_START = NGROUPS - GPW
NSUB = 16
STAGE_ROWS = 632


def _mm_body(x_ref, w_ref, o_ref):
    o_ref[...] = jnp.dot(
        x_ref[...].astype(jnp.bfloat16),
        w_ref[...].astype(jnp.bfloat16),
        preferred_element_type=jnp.float32,
    )


def _matmul(X, W):
    BM = 400
    return pl.pallas_call(
        _mm_body,
        grid=(N // BM,),
        in_specs=[
            pl.BlockSpec((BM, F), lambda i: (i, 0)),
            pl.BlockSpec((F, F), lambda i: (0, 0)),
        ],
        out_specs=pl.BlockSpec((BM, F), lambda i: (i, 0)),
        out_shape=jax.ShapeDtypeStruct((N, F), jnp.float32),
    )(X, W)


HGPW = GPW // 2


def _agg_body(xw_hbm, ci_hbm, val_hbm, out_hbm, shared, idx_v, val_v,
              rb0, rb1, ob0, ob1, sem0, sem1, semo0, semo1):
    wid = lax.axis_index("s") * 2 + lax.axis_index("c")
    sid = lax.axis_index("s")
    r0 = (sid * (N // NSUB)) // 8 * 8
    pltpu.sync_copy(xw_hbm.at[pl.ds(r0, STAGE_ROWS), :],
                    shared.at[pl.ds(r0, STAGE_ROWS), :])
    start_g = jnp.minimum(wid * GPW, MAX_START)
    plsc.subcore_barrier()

    def start_gather(g, rb, sem):
        idx_slice = idx_v.at[pl.ds(g * IPG, IPG)]
        return pltpu.async_copy(shared.at[idx_slice], rb, sem)

    def wait_gather(rb, sem):
        pltpu.make_async_copy(shared.at[idx_v.at[pl.ds(0, IPG)]], rb, sem).wait()

    def half(h):
        base_e = (start_g + h * HGPW) * IPG
        pltpu.sync_copy(ci_hbm.at[pl.ds(base_e, HGPW * IPG)], idx_v)
        pltpu.sync_copy(val_hbm.at[pl.ds(base_e, HGPW * IPG)], val_v)

        def compute(g, rb, ob):
            def node_body(nn, carry2):
                e0 = g * IPG + nn * DEG
                v0 = val_v[pl.ds(e0, 16)]
                v1 = val_v[pl.ds(e0 + 16, 16)]
                rr = nn * DEG
                accs = [jnp.zeros((16,), jnp.float32) for _ in range(8)]
                for j in range(DEG):
                    v = (v0 if j < 16 else v1)[j % 16]
                    for c in range(8):
                        accs[c] = accs[c] + v * rb[rr + j, pl.ds(c * 16, 16)]
                for c in range(8):
                    ob[nn, pl.ds(c * 16, 16)] = accs[c]
                return carry2

            lax.fori_loop(0, NPG, node_body, 0)
            return pltpu.async_copy(
                ob,
                out_hbm.at[pl.ds((start_g + h * HGPW + g) * NPG, NPG), :],
                semo0 if ob is ob0 else semo1)

        start_gather(0, rb0, sem0)

        def body(t, carry):
            g = 2 * t
            start_gather(g + 1, rb1, sem1)
            wait_gather(rb0, sem0)
            cp0 = compute(g, rb0, ob0)
            start_gather(g + 2, rb0, sem0)
            wait_gather(rb1, sem1)
            cp1 = compute(g + 1, rb1, ob1)
            cp0.wait()
            cp1.wait()
            return carry

        lax.fori_loop(0, HGPW // 2 - 1, body, 0)
        g = HGPW - 2
        wait_gather(rb0, sem0)
        cp0 = compute(g, rb0, ob0)
        start_gather(g + 1, rb1, sem1)
        wait_gather(rb1, sem1)
        cp1 = compute(g + 1, rb1, ob1)
        cp0.wait()
        cp1.wait()

    half(0)
    half(1)


def _aggregate(XW, col_idx, vals):
    mesh = plsc.VectorSubcoreMesh(core_axis_name="c", subcore_axis_name="s")
    f = pl.kernel(
        _agg_body,
        out_type=jax.ShapeDtypeStruct((N, F), jnp.float32),
        mesh=mesh,
        scratch_types=[
            pltpu.VMEM_SHARED((N, F), jnp.float32),
            pltpu.VMEM((HGPW * IPG,), jnp.int32),
            pltpu.VMEM((HGPW * IPG,), jnp.float32),
            pltpu.VMEM((IPG, F), jnp.float32),
            pltpu.VMEM((IPG, F), jnp.float32),
            pltpu.VMEM((NPG, F), jnp.float32),
            pltpu.VMEM((NPG, F), jnp.float32),
            pltpu.SemaphoreType.DMA,
            pltpu.SemaphoreType.DMA,
            pltpu.SemaphoreType.DMA,
            pltpu.SemaphoreType.DMA,
        ],
    )
    return f(XW, col_idx, vals)


def kernel(row_ptr, col_idx, values, X, num_neighbors, W):
    XW = _matmul(X, W)
    return _aggregate(XW, col_idx, values)


# speedup vs baseline: 1.0026x; 1.0026x over previous
import jax
import jax.numpy as jnp
from jax import lax
from jax.experimental import pallas as pl
from jax.experimental.pallas import tpu as pltpu
from jax.experimental.pallas import tpu_sc as plsc

N = 10000
DEG = 32
F = 128
NPG = 4
IPG = NPG * DEG
NGROUPS = N // NPG
NWORKERS = 32
GPW = 80
MAX_START = NGROUPS - GPW
NSUB = 16
STAGE_ROWS = 632


def _mm_body(x_ref, w_ref, o_ref):
    o_ref[...] = jnp.dot(x_ref[...], w_ref[...], preferred_element_type=jnp.float32)


def _matmul(X, W):
    BM = 400
    return pl.pallas_call(
        _mm_body,
        grid=(N // BM,),
        in_specs=[
            pl.BlockSpec((BM, F), lambda i: (i, 0)),
            pl.BlockSpec((F, F), lambda i: (0, 0)),
        ],
        out_specs=pl.BlockSpec((BM, F), lambda i: (i, 0)),
        out_shape=jax.ShapeDtypeStruct((N, F), jnp.float32),
    )(X, W)


HGPW = GPW // 2


def _agg_body(xw_hbm, ci_hbm, val_hbm, out_hbm, shared, idx_v, val_v,
              rb0, rb1, ob0, ob1, sem0, sem1, semo0, semo1):
    wid = lax.axis_index("s") * 2 + lax.axis_index("c")
    sid = lax.axis_index("s")
    r0 = (sid * (N // NSUB)) // 8 * 8
    pltpu.sync_copy(xw_hbm.at[pl.ds(r0, STAGE_ROWS), :],
                    shared.at[pl.ds(r0, STAGE_ROWS), :])
    start_g = jnp.minimum(wid * GPW, MAX_START)
    plsc.subcore_barrier()

    def start_gather(g, rb, sem):
        idx_slice = idx_v.at[pl.ds(g * IPG, IPG)]
        return pltpu.async_copy(shared.at[idx_slice], rb, sem)

    def wait_gather(rb, sem):
        pltpu.make_async_copy(shared.at[idx_v.at[pl.ds(0, IPG)]], rb, sem).wait()

    def half(h):
        base_e = (start_g + h * HGPW) * IPG
        pltpu.sync_copy(ci_hbm.at[pl.ds(base_e, HGPW * IPG)], idx_v)
        pltpu.sync_copy(val_hbm.at[pl.ds(base_e, HGPW * IPG)], val_v)

        def compute(g, rb, ob):
            def node_body(nn, carry2):
                e0 = g * IPG + nn * DEG
                v0 = val_v[pl.ds(e0, 16)]
                v1 = val_v[pl.ds(e0 + 16, 16)]
                rr = nn * DEG
                accs = [jnp.zeros((16,), jnp.float32) for _ in range(8)]
                for j in range(DEG):
                    v = (v0 if j < 16 else v1)[j % 16]
                    for c in range(8):
                        accs[c] = accs[c] + v * rb[rr + j, pl.ds(c * 16, 16)]
                for c in range(8):
                    ob[nn, pl.ds(c * 16, 16)] = accs[c]
                return carry2

            lax.fori_loop(0, NPG, node_body, 0)
            return pltpu.async_copy(
                ob,
                out_hbm.at[pl.ds((start_g + h * HGPW + g) * NPG, NPG), :],
                semo0 if ob is ob0 else semo1)

        start_gather(0, rb0, sem0)

        def body(t, carry):
            g = 2 * t
            start_gather(g + 1, rb1, sem1)
            wait_gather(rb0, sem0)
            cp0 = compute(g, rb0, ob0)
            start_gather(g + 2, rb0, sem0)
            wait_gather(rb1, sem1)
            cp1 = compute(g + 1, rb1, ob1)
            cp0.wait()
            cp1.wait()
            return carry

        lax.fori_loop(0, HGPW // 2 - 1, body, 0)
        g = HGPW - 2
        wait_gather(rb0, sem0)
        cp0 = compute(g, rb0, ob0)
        start_gather(g + 1, rb1, sem1)
        wait_gather(rb1, sem1)
        cp1 = compute(g + 1, rb1, ob1)
        cp0.wait()
        cp1.wait()

    half(0)
    half(1)


def _aggregate(XW, col_idx, vals):
    mesh = plsc.VectorSubcoreMesh(core_axis_name="c", subcore_axis_name="s")
    f = pl.kernel(
        _agg_body,
        out_type=jax.ShapeDtypeStruct((N, F), jnp.float32),
        mesh=mesh,
        scratch_types=[
            pltpu.VMEM_SHARED((N, F), jnp.float32),
            pltpu.VMEM((HGPW * IPG,), jnp.int32),
            pltpu.VMEM((HGPW * IPG,), jnp.float32),
            pltpu.VMEM((IPG, F), jnp.float32),
            pltpu.VMEM((IPG, F), jnp.float32),
            pltpu.VMEM((NPG, F), jnp.float32),
            pltpu.VMEM((NPG, F), jnp.float32),
            pltpu.SemaphoreType.DMA,
            pltpu.SemaphoreType.DMA,
            pltpu.SemaphoreType.DMA,
            pltpu.SemaphoreType.DMA,
        ],
    )
    return f(XW, col_idx, vals)


def kernel(row_ptr, col_idx, values, X, num_neighbors, W):
    XW = _matmul(X, W)
    return _aggregate(XW, col_idx, values)


# speedup vs baseline: 1.0250x; 1.0224x over previous
import jax
import jax.numpy as jnp
from jax import lax
from jax.experimental import pallas as pl
from jax.experimental.pallas import tpu as pltpu
from jax.experimental.pallas import tpu_sc as plsc

N = 10000
DEG = 32
F = 128
NPG = 4
IPG = NPG * DEG
NGROUPS = N // NPG
NWORKERS = 32
GPW = 80
MAX_START = NGROUPS - GPW
NSUB = 16
STAGE_ROWS = 632


def _mm_body(x_ref, w_ref, o_ref):
    o_ref[...] = jnp.dot(x_ref[...], w_ref[...], preferred_element_type=jnp.float32)


def _matmul(X, W):
    BM = 400
    return pl.pallas_call(
        _mm_body,
        grid=(N // BM,),
        in_specs=[
            pl.BlockSpec((BM, F), lambda i: (i, 0)),
            pl.BlockSpec((F, F), lambda i: (0, 0)),
        ],
        out_specs=pl.BlockSpec((BM, F), lambda i: (i, 0)),
        out_shape=jax.ShapeDtypeStruct((N, F), jnp.float32),
    )(X, W)


HGPW = GPW // 2


def _agg_body(xw_hbm, ci_hbm, val_hbm, out_hbm, shared, idx_v, val_v,
              rb0, rb1, ob0, ob1, sem0, sem1, semo0, semo1):
    wid = lax.axis_index("s") * 2 + lax.axis_index("c")
    sid = lax.axis_index("s")
    r0 = (sid * (N // NSUB)) // 8 * 8
    pltpu.sync_copy(xw_hbm.at[pl.ds(r0, STAGE_ROWS), :],
                    shared.at[pl.ds(r0, STAGE_ROWS), :])
    start_g = jnp.minimum(wid * GPW, MAX_START)
    plsc.subcore_barrier()

    def start_gather(g, rb, sem):
        idx_slice = idx_v.at[pl.ds(g * IPG, IPG)]
        return pltpu.async_copy(shared.at[idx_slice], rb, sem)

    def wait_gather(rb, sem):
        pltpu.make_async_copy(shared.at[idx_v.at[pl.ds(0, IPG)]], rb, sem).wait()

    def half(h):
        base_e = (start_g + h * HGPW) * IPG
        pltpu.sync_copy(ci_hbm.at[pl.ds(base_e, HGPW * IPG)], idx_v)
        pltpu.sync_copy(val_hbm.at[pl.ds(base_e, HGPW * IPG)], val_v)

        def compute(g, rb, ob):
            def node_body(nn, carry2):
                e0 = g * IPG + nn * DEG
                v0 = val_v[pl.ds(e0, 16)]
                v1 = val_v[pl.ds(e0 + 16, 16)]
                rr = nn * DEG
                accs = [jnp.zeros((16,), jnp.float32) for _ in range(8)]
                for j in range(DEG):
                    v = (v0 if j < 16 else v1)[j % 16]
                    for c in range(8):
                        accs[c] = accs[c] + v * rb[rr + j, pl.ds(c * 16, 16)]
                for c in range(8):
                    ob[nn, pl.ds(c * 16, 16)] = accs[c]
                return carry2

            lax.fori_loop(0, NPG, node_body, 0)
            return pltpu.async_copy(
                ob,
                out_hbm.at[pl.ds((start_g + h * HGPW + g) * NPG, NPG), :],
                semo0 if ob is ob0 else semo1)

        def out_wait(ob, semo):
            pltpu.make_async_copy(ob, out_hbm.at[pl.ds(0, NPG), :], semo).wait()

        start_gather(0, rb0, sem0)

        def body(t, carry):
            g = 2 * t
            start_gather(g + 1, rb1, sem1)
            wait_gather(rb0, sem0)

            @pl.when(t > 0)
            def _w0():
                out_wait(ob0, semo0)

            compute(g, rb0, ob0)
            start_gather(g + 2, rb0, sem0)
            wait_gather(rb1, sem1)

            @pl.when(t > 0)
            def _w1():
                out_wait(ob1, semo1)

            compute(g + 1, rb1, ob1)
            return carry

        lax.fori_loop(0, HGPW // 2 - 1, body, 0)
        g = HGPW - 2
        wait_gather(rb0, sem0)
        out_wait(ob0, semo0)
        cp0 = compute(g, rb0, ob0)
        start_gather(g + 1, rb1, sem1)
        wait_gather(rb1, sem1)
        out_wait(ob1, semo1)
        cp1 = compute(g + 1, rb1, ob1)
        cp0.wait()
        cp1.wait()

    half(0)
    half(1)


def _aggregate(XW, col_idx, vals):
    mesh = plsc.VectorSubcoreMesh(core_axis_name="c", subcore_axis_name="s")
    f = pl.kernel(
        _agg_body,
        out_type=jax.ShapeDtypeStruct((N, F), jnp.float32),
        mesh=mesh,
        scratch_types=[
            pltpu.VMEM_SHARED((N, F), jnp.float32),
            pltpu.VMEM((HGPW * IPG,), jnp.int32),
            pltpu.VMEM((HGPW * IPG,), jnp.float32),
            pltpu.VMEM((IPG, F), jnp.float32),
            pltpu.VMEM((IPG, F), jnp.float32),
            pltpu.VMEM((NPG, F), jnp.float32),
            pltpu.VMEM((NPG, F), jnp.float32),
            pltpu.SemaphoreType.DMA,
            pltpu.SemaphoreType.DMA,
            pltpu.SemaphoreType.DMA,
            pltpu.SemaphoreType.DMA,
        ],
    )
    return f(XW, col_idx, vals)


def kernel(row_ptr, col_idx, values, X, num_neighbors, W):
    XW = _matmul(X, W)
    return _aggregate(XW, col_idx, values)


# speedup vs baseline: 1.0440x; 1.0185x over previous
import jax
import jax.numpy as jnp
from jax import lax
from jax.experimental import pallas as pl
from jax.experimental.pallas import tpu as pltpu
from jax.experimental.pallas import tpu_sc as plsc

N = 10000
DEG = 32
F = 128
NPG = 4
IPG = NPG * DEG
NGROUPS = N // NPG
NWORKERS = 32
GPW = 80
MAX_START = NGROUPS - GPW
NSUB = 16
STAGE_ROWS = 632


def _mm_body(x_ref, w_ref, o_ref):
    o_ref[...] = jnp.dot(x_ref[...], w_ref[...], preferred_element_type=jnp.float32)


def _matmul(X, W):
    BM = 400
    return pl.pallas_call(
        _mm_body,
        grid=(N // BM,),
        in_specs=[
            pl.BlockSpec((BM, F), lambda i: (i, 0)),
            pl.BlockSpec((F, F), lambda i: (0, 0)),
        ],
        out_specs=pl.BlockSpec((BM, F), lambda i: (i, 0)),
        out_shape=jax.ShapeDtypeStruct((N, F), jnp.float32),
    )(X, W)


HGPW = GPW // 2


def _agg_body(xw_hbm, ci_hbm, val_hbm, out_hbm, shared, idx_v, val_v,
              rb0, rb1, ob0, ob1, sem0, sem1, semo0, semo1):
    wid = lax.axis_index("s") * 2 + lax.axis_index("c")
    sid = lax.axis_index("s")
    start_g = jnp.minimum(wid * GPW, MAX_START)

    def stage_half(h):
        base_e = (start_g + h * HGPW) * IPG
        return (
            pltpu.async_copy(ci_hbm.at[pl.ds(base_e, HGPW * IPG)], idx_v, semo0),
            pltpu.async_copy(val_hbm.at[pl.ds(base_e, HGPW * IPG)], val_v, semo1),
        )

    stage0 = stage_half(0)
    r0 = (sid * (N // NSUB)) // 8 * 8
    pltpu.sync_copy(xw_hbm.at[pl.ds(r0, STAGE_ROWS), :],
                    shared.at[pl.ds(r0, STAGE_ROWS), :])
    plsc.subcore_barrier()

    def start_gather(g, rb, sem):
        idx_slice = idx_v.at[pl.ds(g * IPG, IPG)]
        return pltpu.async_copy(shared.at[idx_slice], rb, sem)

    def wait_gather(rb, sem):
        pltpu.make_async_copy(shared.at[idx_v.at[pl.ds(0, IPG)]], rb, sem).wait()

    def half(h, stage_cps):
        stage_cps[0].wait()
        stage_cps[1].wait()

        def compute(g, rb, ob):
            def node_body(nn, carry2):
                e0 = g * IPG + nn * DEG
                v0 = val_v[pl.ds(e0, 16)]
                v1 = val_v[pl.ds(e0 + 16, 16)]
                rr = nn * DEG
                accs = [jnp.zeros((16,), jnp.float32) for _ in range(8)]
                for j in range(DEG):
                    v = (v0 if j < 16 else v1)[j % 16]
                    for c in range(8):
                        accs[c] = accs[c] + v * rb[rr + j, pl.ds(c * 16, 16)]
                for c in range(8):
                    ob[nn, pl.ds(c * 16, 16)] = accs[c]
                return carry2

            lax.fori_loop(0, NPG, node_body, 0)
            return pltpu.async_copy(
                ob,
                out_hbm.at[pl.ds((start_g + h * HGPW + g) * NPG, NPG), :],
                semo0 if ob is ob0 else semo1)

        def out_wait(ob, semo):
            pltpu.make_async_copy(ob, out_hbm.at[pl.ds(0, NPG), :], semo).wait()

        start_gather(0, rb0, sem0)

        def body(t, carry):
            g = 2 * t
            start_gather(g + 1, rb1, sem1)
            wait_gather(rb0, sem0)

            @pl.when(t > 0)
            def _w0():
                out_wait(ob0, semo0)

            compute(g, rb0, ob0)
            start_gather(g + 2, rb0, sem0)
            wait_gather(rb1, sem1)

            @pl.when(t > 0)
            def _w1():
                out_wait(ob1, semo1)

            compute(g + 1, rb1, ob1)
            return carry

        lax.fori_loop(0, HGPW // 2 - 1, body, 0)
        g = HGPW - 2
        wait_gather(rb0, sem0)
        out_wait(ob0, semo0)
        cp0 = compute(g, rb0, ob0)
        start_gather(g + 1, rb1, sem1)
        wait_gather(rb1, sem1)
        out_wait(ob1, semo1)
        cp1 = compute(g + 1, rb1, ob1)
        cp0.wait()
        cp1.wait()

    half(0, stage0)
    half(1, stage_half(1))


def _aggregate(XW, col_idx, vals):
    mesh = plsc.VectorSubcoreMesh(core_axis_name="c", subcore_axis_name="s")
    f = pl.kernel(
        _agg_body,
        out_type=jax.ShapeDtypeStruct((N, F), jnp.float32),
        mesh=mesh,
        scratch_types=[
            pltpu.VMEM_SHARED((N, F), jnp.float32),
            pltpu.VMEM((HGPW * IPG,), jnp.int32),
            pltpu.VMEM((HGPW * IPG,), jnp.float32),
            pltpu.VMEM((IPG, F), jnp.float32),
            pltpu.VMEM((IPG, F), jnp.float32),
            pltpu.VMEM((NPG, F), jnp.float32),
            pltpu.VMEM((NPG, F), jnp.float32),
            pltpu.SemaphoreType.DMA,
            pltpu.SemaphoreType.DMA,
            pltpu.SemaphoreType.DMA,
            pltpu.SemaphoreType.DMA,
        ],
    )
    return f(XW, col_idx, vals)


def kernel(row_ptr, col_idx, values, X, num_neighbors, W):
    XW = _matmul(X, W)
    return _aggregate(XW, col_idx, values)


# speedup vs baseline: 1.0580x; 1.0134x over previous
import jax
import jax.numpy as jnp
from jax import lax
from jax.experimental import pallas as pl
from jax.experimental.pallas import tpu as pltpu
from jax.experimental.pallas import tpu_sc as plsc

N = 10000
DEG = 32
F = 128
NPG = 4
IPG = NPG * DEG
NGROUPS = N // NPG
NWORKERS = 32
GPW = 80
MAX_START = NGROUPS - GPW
NSUB = 16
STAGE_ROWS = 632


def _mm_body(x_ref, w_ref, o_ref):
    o_ref[...] = jnp.dot(x_ref[...], w_ref[...], preferred_element_type=jnp.float32)


def _matmul(X, W):
    BM = 400
    return pl.pallas_call(
        _mm_body,
        grid=(N // BM,),
        in_specs=[
            pl.BlockSpec((BM, F), lambda i: (i, 0)),
            pl.BlockSpec((F, F), lambda i: (0, 0)),
        ],
        out_specs=pl.BlockSpec((BM, F), lambda i: (i, 0)),
        out_shape=jax.ShapeDtypeStruct((N, F), jnp.float32),
    )(X, W)


HGPW = GPW // 2


def _agg_body(xw_hbm, ci_hbm, val_hbm, out_hbm, shared, idx_v, val_v,
              rb0, rb1, ob0, ob1, sem0, sem1, semo0, semo1):
    wid = lax.axis_index("s") * 2 + lax.axis_index("c")
    sid = lax.axis_index("s")
    start_g = jnp.minimum(wid * GPW, MAX_START)

    def stage_half(h):
        base_e = (start_g + h * HGPW) * IPG
        return (
            pltpu.async_copy(ci_hbm.at[pl.ds(base_e, HGPW * IPG)], idx_v, semo0),
            pltpu.async_copy(val_hbm.at[pl.ds(base_e, HGPW * IPG)], val_v, semo1),
        )

    stage0 = stage_half(0)
    r0 = (sid * (N // NSUB)) // 8 * 8
    pltpu.sync_copy(xw_hbm.at[pl.ds(r0, STAGE_ROWS), :],
                    shared.at[pl.ds(r0, STAGE_ROWS), :])
    plsc.subcore_barrier()

    def start_gather(g, rb, sem):
        idx_slice = idx_v.at[pl.ds(g * IPG, IPG)]
        return pltpu.async_copy(shared.at[idx_slice], rb, sem)

    def wait_gather(rb, sem):
        pltpu.make_async_copy(shared.at[idx_v.at[pl.ds(0, IPG)]], rb, sem).wait()

    def half(h, stage_cps):
        stage_cps[0].wait()
        stage_cps[1].wait()

        def compute(g, rb, ob):
            def node_body(nn, carry2):
                e0 = g * IPG + nn * DEG
                v0 = val_v[pl.ds(e0, 16)]
                v1 = val_v[pl.ds(e0 + 16, 16)]
                rr = nn * DEG
                accs = [jnp.zeros((16,), jnp.float32) for _ in range(8)]
                for j in range(DEG):
                    v = (v0 if j < 16 else v1)[j % 16]
                    for c in range(8):
                        accs[c] = accs[c] + v * rb[rr + j, pl.ds(c * 16, 16)]
                for c in range(8):
                    ob[nn, pl.ds(c * 16, 16)] = accs[c]
                return carry2

            lax.fori_loop(0, NPG, node_body, 0)
            return pltpu.async_copy(
                ob,
                out_hbm.at[pl.ds((start_g + h * HGPW + g) * NPG, NPG), :],
                semo0 if ob is ob0 else semo1)

        def out_wait(ob, semo):
            pltpu.make_async_copy(ob, out_hbm.at[pl.ds(0, NPG), :], semo).wait()

        start_gather(0, rb0, sem0)

        def body(t, carry):
            g = 2 * t
            start_gather(g + 1, rb1, sem1)
            wait_gather(rb0, sem0)

            @pl.when(t > 0)
            def _w0():
                out_wait(ob0, semo0)

            compute(g, rb0, ob0)
            start_gather(g + 2, rb0, sem0)
            wait_gather(rb1, sem1)

            @pl.when(t > 0)
            def _w1():
                out_wait(ob1, semo1)

            compute(g + 1, rb1, ob1)
            return carry

        lax.fori_loop(0, HGPW // 2 - 1, body, 0)
        g = HGPW - 2
        start_gather(g + 1, rb1, sem1)
        wait_gather(rb0, sem0)
        out_wait(ob0, semo0)
        cp0 = compute(g, rb0, ob0)
        wait_gather(rb1, sem1)
        out_wait(ob1, semo1)
        cp1 = compute(g + 1, rb1, ob1)
        cp0.wait()
        cp1.wait()

    half(0, stage0)
    half(1, stage_half(1))


def _aggregate(XW, col_idx, vals):
    mesh = plsc.VectorSubcoreMesh(core_axis_name="c", subcore_axis_name="s")
    f = pl.kernel(
        _agg_body,
        out_type=jax.ShapeDtypeStruct((N, F), jnp.float32),
        mesh=mesh,
        scratch_types=[
            pltpu.VMEM_SHARED((N, F), jnp.float32),
            pltpu.VMEM((HGPW * IPG,), jnp.int32),
            pltpu.VMEM((HGPW * IPG,), jnp.float32),
            pltpu.VMEM((IPG, F), jnp.float32),
            pltpu.VMEM((IPG, F), jnp.float32),
            pltpu.VMEM((NPG, F), jnp.float32),
            pltpu.VMEM((NPG, F), jnp.float32),
            pltpu.SemaphoreType.DMA,
            pltpu.SemaphoreType.DMA,
            pltpu.SemaphoreType.DMA,
            pltpu.SemaphoreType.DMA,
        ],
    )
    return f(XW, col_idx, vals)


def kernel(row_ptr, col_idx, values, X, num_neighbors, W):
    XW = _matmul(X, W)
    return _aggregate(XW, col_idx, values)
